# baseline (speedup 1.0000x reference)
import functools

import numpy as np
import jax
import jax.numpy as jnp
from jax import lax
from jax.experimental import pallas as pl
from jax.experimental.pallas import tpu as pltpu
from jax.experimental.pallas import tpu_sc as plsc

_IMAGE = 64
_PSIZE = 8
_STRIDE = 4
_NP = 15
_BATCH = 4
_CHANNELS = 64
_BC = _BATCH * _CHANNELS
_K = _PSIZE * _PSIZE
_NPATCH = _NP * _NP
_KS = _K + 4
_XLEN = _NPATCH * _KS
_XROW = 15312
_NPIX = _IMAGE * _IMAGE
_LANES = 16


def _build_tables():
    idx = np.full((4, _NPIX), _XLEN, dtype=np.int32)
    cnt = np.zeros((_NPIX,), dtype=np.float32)
    for t, (da, db) in enumerate([(0, 0), (0, 1), (1, 0), (1, 1)]):
        for y in range(_IMAGE):
            a = y // _STRIDE - da
            i = y % _STRIDE + _STRIDE * da
            if not 0 <= a < _NP:
                continue
            for x in range(_IMAGE):
                b = x // _STRIDE - db
                j = x % _STRIDE + _STRIDE * db
                if not 0 <= b < _NP:
                    continue
                idx[t, y * _IMAGE + x] = (a * _NP + b) * _KS + i * _PSIZE + j
                cnt[y * _IMAGE + x] += 1.0
    return idx, (1.0 / cnt).astype(np.float32)


_IDX_TAB, _RECIP_TAB = _build_tables()


def _sc_core_counts():
    try:
        info = plsc.get_sparse_core_info()
        return info.num_cores, info.num_subcores
    except Exception:
        return 2, 16


@functools.cache
def _make_sc_kernel():
    nc, ns = _sc_core_counts()
    nw = nc * ns
    rows_per = _BC // nw
    mesh = plsc.VectorSubcoreMesh(core_axis_name="c", subcore_axis_name="s")

    @functools.partial(
        pl.kernel,
        mesh=mesh,
        out_type=jax.ShapeDtypeStruct((_BC, _NPIX), jnp.float32),
        compiler_params=pltpu.CompilerParams(
            needs_layout_passes=False, use_tc_tiling_on_sc=False
        ),
        scratch_types=[
            pltpu.VMEM((_XROW,), jnp.float32),
            pltpu.VMEM((_NPIX,), jnp.float32),
            pltpu.VMEM((4, _NPIX), jnp.int32),
            pltpu.VMEM((_NPIX,), jnp.float32),
        ],
    )
    def k(x_hbm, idx_hbm, recip_hbm, out_hbm, xbuf, obuf, ibuf, rbuf):
        wid = lax.axis_index("s") * nc + lax.axis_index("c")
        pltpu.sync_copy(idx_hbm, ibuf)
        pltpu.sync_copy(recip_hbm, rbuf)

        def body(v, _):
            sl = pl.ds(v * _LANES, _LANES)
            acc = plsc.load_gather(xbuf, [ibuf[0, sl]])
            acc = acc + plsc.load_gather(xbuf, [ibuf[1, sl]])
            acc = acc + plsc.load_gather(xbuf, [ibuf[2, sl]])
            acc = acc + plsc.load_gather(xbuf, [ibuf[3, sl]])
            obuf[sl] = acc * rbuf[sl]
            return 0

        for row in range(rows_per):
            bc = wid * rows_per + row
            pltpu.sync_copy(x_hbm.at[bc], xbuf)
            lax.fori_loop(0, _NPIX // _LANES, body, 0, unroll=4)
            pltpu.sync_copy(obuf, out_hbm.at[bc])

    return k


def kernel(input_data):
    x2 = jnp.pad(input_data, ((0, 0), (0, 0), (0, _KS - _K)))
    x2 = x2.reshape(_BC, _XLEN)
    x2 = jnp.pad(x2, ((0, 0), (0, _XROW - _XLEN)))
    out = _make_sc_kernel()(x2, jnp.asarray(_IDX_TAB), jnp.asarray(_RECIP_TAB))
    return out.reshape(_BATCH, _CHANNELS, _IMAGE, _IMAGE)


# speedup vs baseline: 1.2501x; 1.2501x over previous
import functools

import numpy as np
import jax
import jax.numpy as jnp
from jax import lax
from jax.experimental import pallas as pl
from jax.experimental.pallas import tpu as pltpu
from jax.experimental.pallas import tpu_sc as plsc

_IMAGE = 64
_PSIZE = 8
_STRIDE = 4
_NP = 15
_BATCH = 4
_CHANNELS = 64
_BC = _BATCH * _CHANNELS
_K = _PSIZE * _PSIZE
_NPATCH = _NP * _NP
_XLEN = _NPATCH * _K
_PAD = 16
_XLEN_PAD = _XLEN + _PAD
_NPIX = _IMAGE * _IMAGE
_LANES = 16


def _build_tables():
    idx = np.full((4, _NPIX), _XLEN, dtype=np.int32)
    cnt = np.zeros((_NPIX,), dtype=np.float32)
    for t, (da, db) in enumerate([(0, 0), (0, 1), (1, 0), (1, 1)]):
        for y in range(_IMAGE):
            a = y // _STRIDE - da
            i = y % _STRIDE + _STRIDE * da
            if not 0 <= a < _NP:
                continue
            for x in range(_IMAGE):
                b = x // _STRIDE - db
                j = x % _STRIDE + _STRIDE * db
                if not 0 <= b < _NP:
                    continue
                idx[t, y * _IMAGE + x] = (a * _NP + b) * _K + i * _PSIZE + j
                cnt[y * _IMAGE + x] += 1.0
    return idx.reshape(-1), (1.0 / cnt).astype(np.float32)


_IDX_TAB, _RECIP_TAB = _build_tables()


def _sc_core_counts():
    try:
        info = plsc.get_sparse_core_info()
        return info.num_cores, info.num_subcores
    except Exception:
        return 2, 16


@functools.cache
def _make_sc_kernel():
    nc, ns = _sc_core_counts()
    nw = nc * ns
    rows_per = _BC // nw
    mesh = plsc.VectorSubcoreMesh(core_axis_name="c", subcore_axis_name="s")

    @functools.partial(
        pl.kernel,
        mesh=mesh,
        out_type=jax.ShapeDtypeStruct((_BC * _NPIX,), jnp.float32),
        compiler_params=pltpu.CompilerParams(
            needs_layout_passes=False, use_tc_tiling_on_sc=False
        ),
        scratch_types=[
            pltpu.VMEM((_XLEN_PAD,), jnp.float32),
            pltpu.VMEM((_NPIX,), jnp.float32),
            pltpu.VMEM((4 * _NPIX,), jnp.int32),
            pltpu.VMEM((_NPIX,), jnp.float32),
        ],
    )
    def k(x_hbm, idx_hbm, recip_hbm, out_hbm, xbuf, obuf, ibuf, rbuf):
        wid = lax.axis_index("s") * nc + lax.axis_index("c")
        pltpu.sync_copy(idx_hbm, ibuf)
        pltpu.sync_copy(recip_hbm, rbuf)
        xbuf[pl.ds(_XLEN, _PAD)] = jnp.zeros((_PAD,), jnp.float32)

        def body(v, _):
            o = v * _LANES
            sl = pl.ds(o, _LANES)
            acc = plsc.load_gather(xbuf, [ibuf[pl.ds(o, _LANES)]])
            acc = acc + plsc.load_gather(xbuf, [ibuf[pl.ds(_NPIX + o, _LANES)]])
            acc = acc + plsc.load_gather(
                xbuf, [ibuf[pl.ds(2 * _NPIX + o, _LANES)]]
            )
            acc = acc + plsc.load_gather(
                xbuf, [ibuf[pl.ds(3 * _NPIX + o, _LANES)]]
            )
            obuf[sl] = acc * rbuf[sl]
            return 0

        for row in range(rows_per):
            bc = wid * rows_per + row
            pltpu.sync_copy(
                x_hbm.at[pl.ds(bc * _XLEN, _XLEN)], xbuf.at[pl.ds(0, _XLEN)]
            )
            lax.fori_loop(0, _NPIX // _LANES, body, 0, unroll=4)
            pltpu.sync_copy(obuf, out_hbm.at[pl.ds(bc * _NPIX, _NPIX)])

    return k


def kernel(input_data):
    x1 = input_data.reshape(_BC * _XLEN)
    out = _make_sc_kernel()(x1, jnp.asarray(_IDX_TAB), jnp.asarray(_RECIP_TAB))
    return out.reshape(_BATCH, _CHANNELS, _IMAGE, _IMAGE)


# speedup vs baseline: 1.5385x; 1.2307x over previous
import functools

import jax
import jax.numpy as jnp
from jax import lax
from jax.experimental import pallas as pl
from jax.experimental.pallas import tpu as pltpu
from jax.experimental.pallas import tpu_sc as plsc

_IMAGE = 64
_PSIZE = 8
_STRIDE = 4
_NP = 15
_BATCH = 4
_CHANNELS = 64
_BC = _BATCH * _CHANNELS
_NPATCH = _NP * _NP
_NPIX = _IMAGE * _IMAGE
_HALF = _NPIX // 2
_LANES = 16


def _sc_core_counts():
    try:
        info = plsc.get_sparse_core_info()
        return info.num_cores, info.num_subcores
    except Exception:
        return 2, 16


@functools.cache
def _make_sc_kernel():
    nc, ns = _sc_core_counts()
    mesh = plsc.VectorSubcoreMesh(core_axis_name="c", subcore_axis_name="s")

    @functools.partial(
        pl.kernel,
        mesh=mesh,
        out_type=jax.ShapeDtypeStruct((_NPIX, _BC), jnp.float32),
        compiler_params=pltpu.CompilerParams(
            needs_layout_passes=False, use_tc_tiling_on_sc=False
        ),
        scratch_types=[
            pltpu.VMEM((_NP, 8, 8, _LANES), jnp.float32),
            pltpu.VMEM((_HALF, _LANES), jnp.float32),
        ],
    )
    def k(x_hbm, out_hbm, slab, obuf):
        wid = lax.axis_index("s") * nc + lax.axis_index("c")
        g = wid // 2
        h = wid % 2
        ghi = g // 8
        glo = g % 8

        zeros = jnp.zeros((_LANES,), jnp.float32)

        def zbody(p_, _):
            obuf[p_, :] = zeros
            return 0

        lax.fori_loop(0, _HALF, zbody, 0, unroll=8)

        def emit_half(y0, a0):
            for ablk in range(8):
                a = a0 + ablk
                istart = max(0, y0 - 4 * a)
                iend = min(8, y0 + 32 - 4 * a)
                pltpu.sync_copy(
                    x_hbm.at[
                        pl.ds(a * _NP, _NP),
                        :,
                        ghi,
                        :,
                        pl.ds(glo * _LANES, _LANES),
                    ],
                    slab,
                )

                def bbody(b, _, a=a, istart=istart, iend=iend, y0=y0):
                    for i in range(istart, iend):
                        row = (4 * a + i - y0) * _IMAGE
                        for j in range(_PSIZE):
                            plsc.addupdate(
                                obuf.at[row + 4 * b + j], slab[b, i, j, :]
                            )
                    return 0

                lax.fori_loop(0, _NP, bbody, 0)

            def srow(yl, _):
                y = y0 + yl
                ry = jnp.where(
                    (y < 4) | (y >= 60), jnp.float32(0.5), jnp.float32(0.25)
                )
                ry2 = ry * jnp.float32(2.0)
                base = yl * _IMAGE
                for x in range(_IMAGE):
                    f = ry2 if (x < 4 or x >= 60) else ry
                    obuf[base + x, :] = obuf[base + x, :] * f
                return 0

            lax.fori_loop(0, 32, srow, 0)

            pltpu.sync_copy(
                obuf,
                out_hbm.at[
                    pl.ds(y0 * _IMAGE, _HALF), pl.ds(g * _LANES, _LANES)
                ],
            )

        @pl.when(h == 0)
        def _():
            emit_half(0, 0)

        @pl.when(h == 1)
        def _():
            emit_half(32, 7)

    return k


def kernel(input_data):
    x5 = (
        input_data.transpose(1, 2, 0)
        .reshape(_NPATCH, 8, 8, 2, 128)
        .transpose(0, 1, 3, 2, 4)
    )
    out = _make_sc_kernel()(x5)
    return out.reshape(_IMAGE, _IMAGE, _BATCH, _CHANNELS).transpose(2, 3, 0, 1)


# speedup vs baseline: 2.1111x; 1.3722x over previous
import functools

import jax
import jax.numpy as jnp
from jax import lax
from jax.experimental import pallas as pl
from jax.experimental.pallas import tpu as pltpu
from jax.experimental.pallas import tpu_sc as plsc

_IMAGE = 64
_PSIZE = 8
_NP = 15
_BATCH = 4
_CHANNELS = 64
_BC = _BATCH * _CHANNELS
_NPATCH = _NP * _NP
_NPIX = _IMAGE * _IMAGE
_HALF = _NPIX // 2
_LANES = 16


def _sc_core_counts():
    try:
        info = plsc.get_sparse_core_info()
        return info.num_cores, info.num_subcores
    except Exception:
        return 2, 16


@functools.cache
def _make_sc_kernel():
    nc, ns = _sc_core_counts()
    mesh = plsc.VectorSubcoreMesh(core_axis_name="c", subcore_axis_name="s")

    @functools.partial(
        pl.kernel,
        mesh=mesh,
        out_type=jax.ShapeDtypeStruct((_NPIX, _BC), jnp.float32),
        compiler_params=pltpu.CompilerParams(
            needs_layout_passes=False, use_tc_tiling_on_sc=False
        ),
        scratch_types=[
            pltpu.VMEM((_NP, 8, 8, _LANES), jnp.float32),
            pltpu.VMEM((_HALF, _LANES), jnp.float32),
        ],
    )
    def k(x_hbm, out_hbm, slab, obuf):
        wid = lax.axis_index("s") * nc + lax.axis_index("c")
        g = wid // 2
        h = wid % 2
        ghi = g // 8
        glo = g % 8

        zeros = jnp.zeros((_LANES,), jnp.float32)

        def zbody(p_, _):
            obuf[p_, :] = zeros
            return 0

        lax.fori_loop(0, _HALF, zbody, 0, unroll=8)

        def load_slab(a):
            pltpu.sync_copy(
                x_hbm.at[
                    pl.ds(a * _NP, _NP),
                    :,
                    ghi,
                    :,
                    pl.ds(glo * _LANES, _LANES),
                ],
                slab,
            )

        def accum(b, base, i_list, ry_vecs):
            for blk in range(0, len(i_list), 2):
                pair = i_list[blk:blk + 2]
                vals = [
                    slab[b, i, j, :] * ry_vecs[i]
                    for i in pair
                    for j in range(_PSIZE)
                ]
                n = 0
                for i in pair:
                    for j in range(_PSIZE):
                        v = vals[n]
                        n += 1
                        if isinstance(b, int) and (
                            (b == 0 and j < 4) or (b == _NP - 1 and j >= 4)
                        ):
                            v = v + v
                        plsc.addupdate(
                            obuf.at[base + i * _IMAGE + 4 * b + j], v
                        )

        def emit_block(a, hh, i_list, border_rows):
            load_slab(a)
            base = (4 * a) * _IMAGE - hh * _HALF
            ry_vecs = {}
            for i in i_list:
                if border_rows:
                    y = 4 * a + i
                    ry = jnp.where(
                        (y < 4) | (y >= 60),
                        jnp.float32(0.5),
                        jnp.float32(0.25),
                    )
                else:
                    ry = jnp.float32(0.25)
                ry_vecs[i] = jnp.broadcast_to(ry, (_LANES,))

            accum(0, base, i_list, ry_vecs)

            def bbody(b, _):
                accum(b, base, i_list, ry_vecs)
                return 0

            lax.fori_loop(1, _NP - 1, bbody, 0)
            accum(_NP - 1, base, i_list, ry_vecs)

        @pl.when(h == 0)
        def _():
            emit_block(7, 0, [0, 1, 2, 3], border_rows=False)

        @pl.when(h == 1)
        def _():
            emit_block(7, 1, [4, 5, 6, 7], border_rows=False)

        def main_body(ablk, _):
            emit_block(ablk + 8 * h, h, list(range(8)), border_rows=True)
            return 0

        lax.fori_loop(0, 7, main_body, 0)

        pltpu.sync_copy(
            obuf,
            out_hbm.at[pl.ds(h * _HALF, _HALF), pl.ds(g * _LANES, _LANES)],
        )

    return k


def kernel(input_data):
    x5 = (
        input_data.transpose(1, 2, 0)
        .reshape(_NPATCH, 8, 8, 2, 128)
        .transpose(0, 1, 3, 2, 4)
    )
    out = _make_sc_kernel()(x5)
    return out.reshape(_IMAGE, _IMAGE, _BATCH, _CHANNELS).transpose(2, 3, 0, 1)


# speedup vs baseline: 2.4024x; 1.1380x over previous
import functools

import jax
import jax.numpy as jnp
from jax import lax
from jax.experimental import pallas as pl
from jax.experimental.pallas import tpu as pltpu
from jax.experimental.pallas import tpu_sc as plsc

_IMAGE = 64
_PSIZE = 8
_NP = 15
_BATCH = 4
_CHANNELS = 64
_BC = _BATCH * _CHANNELS
_NPATCH = _NP * _NP
_NPIX = _IMAGE * _IMAGE
_HALF = _NPIX // 2
_LANES = 16


def _sc_core_counts():
    try:
        info = plsc.get_sparse_core_info()
        return info.num_cores, info.num_subcores
    except Exception:
        return 2, 16


@functools.cache
def _make_sc_kernel():
    nc, ns = _sc_core_counts()
    mesh = plsc.VectorSubcoreMesh(core_axis_name="c", subcore_axis_name="s")

    @functools.partial(
        pl.kernel,
        mesh=mesh,
        out_type=jax.ShapeDtypeStruct((_NPIX, _BC), jnp.float32),
        compiler_params=pltpu.CompilerParams(
            needs_layout_passes=False, use_tc_tiling_on_sc=False
        ),
        scratch_types=[
            pltpu.VMEM((2, _NP, 8, 8, _LANES), jnp.float32),
            pltpu.VMEM((_HALF, _LANES), jnp.float32),
            pltpu.SemaphoreType.DMA((2,)),
        ],
    )
    def k(x_hbm, out_hbm, slab2, obuf, sem):
        wid = lax.axis_index("s") * nc + lax.axis_index("c")
        g = wid // 2
        h = wid % 2
        ghi = g // 8
        glo = g % 8

        zeros = jnp.zeros((_LANES,), jnp.float32)

        def slab_copy(a, buf):
            return pltpu.make_async_copy(
                x_hbm.at[
                    pl.ds(a * _NP, _NP),
                    :,
                    ghi,
                    :,
                    pl.ds(glo * _LANES, _LANES),
                ],
                slab2.at[buf],
                sem.at[buf],
            )

        slab_copy(7, 0).start()
        slab_copy(8 * h, 1).start()

        def zbody(p_, _):
            obuf[p_, :] = zeros
            return 0

        lax.fori_loop(0, _HALF, zbody, 0, unroll=8)

        def accum(slab, b, base, i_list, ry_vecs):
            for blk in range(0, len(i_list), 2):
                pair = i_list[blk:blk + 2]
                vals = [
                    slab[b, i, j, :] * ry_vecs[i]
                    for i in pair
                    for j in range(_PSIZE)
                ]
                n = 0
                for i in pair:
                    for j in range(_PSIZE):
                        v = vals[n]
                        n += 1
                        if isinstance(b, int) and (
                            (b == 0 and j < 4) or (b == _NP - 1 and j >= 4)
                        ):
                            v = v + v
                        plsc.addupdate(
                            obuf.at[base + i * _IMAGE + 4 * b + j], v
                        )

        def emit_block(slab, a, hh, i_list, border_rows):
            base = (4 * a) * _IMAGE - hh * _HALF
            ry_vecs = {}
            for i in i_list:
                if border_rows:
                    y = 4 * a + i
                    ry = jnp.where(
                        (y < 4) | (y >= 60),
                        jnp.float32(0.5),
                        jnp.float32(0.25),
                    )
                else:
                    ry = jnp.float32(0.25)
                ry_vecs[i] = jnp.broadcast_to(ry, (_LANES,))

            accum(slab, 0, base, i_list, ry_vecs)

            def bbody(b, _):
                accum(slab, b, base, i_list, ry_vecs)
                return 0

            lax.fori_loop(1, _NP - 1, bbody, 0)
            accum(slab, _NP - 1, base, i_list, ry_vecs)

        slab_copy(7, 0).wait()

        @pl.when(h == 0)
        def _():
            emit_block(slab2.at[0], 7, 0, [0, 1, 2, 3], border_rows=False)

        @pl.when(h == 1)
        def _():
            emit_block(slab2.at[0], 7, 1, [4, 5, 6, 7], border_rows=False)

        slab_copy(8 * h + 1, 0).start()

        def main_body(s, _):
            a = 8 * h + s - 1
            buf = s % 2
            slab_copy(a, buf).wait()
            emit_block(slab2.at[buf], a, h, list(range(8)), border_rows=True)

            @pl.when(s <= 5)
            def _():
                slab_copy(a + 2, buf).start()

            return 0

        lax.fori_loop(1, 8, main_body, 0)

        pltpu.sync_copy(
            obuf,
            out_hbm.at[pl.ds(h * _HALF, _HALF), pl.ds(g * _LANES, _LANES)],
        )

    return k


def kernel(input_data):
    x5 = (
        input_data.transpose(1, 2, 0)
        .reshape(_NPATCH, 8, 8, 2, 128)
        .transpose(0, 1, 3, 2, 4)
    )
    out = _make_sc_kernel()(x5)
    return out.reshape(_IMAGE, _IMAGE, _BATCH, _CHANNELS).transpose(2, 3, 0, 1)


# speedup vs baseline: 2.6427x; 1.1000x over previous
import functools

import jax
import jax.numpy as jnp
from jax import lax
from jax.experimental import pallas as pl
from jax.experimental.pallas import tpu as pltpu
from jax.experimental.pallas import tpu_sc as plsc

_IMAGE = 64
_PSIZE = 8
_NP = 15
_BATCH = 4
_CHANNELS = 64
_BC = _BATCH * _CHANNELS
_NPATCH = _NP * _NP
_NPIX = _IMAGE * _IMAGE
_HALF = _NPIX // 2
_LANES = 16


def _sc_core_counts():
    try:
        info = plsc.get_sparse_core_info()
        return info.num_cores, info.num_subcores
    except Exception:
        return 2, 16


@functools.cache
def _make_sc_kernel():
    nc, ns = _sc_core_counts()
    mesh = plsc.VectorSubcoreMesh(core_axis_name="c", subcore_axis_name="s")

    @functools.partial(
        pl.kernel,
        mesh=mesh,
        out_type=jax.ShapeDtypeStruct((_BC, _NPIX), jnp.float32),
        compiler_params=pltpu.CompilerParams(
            needs_layout_passes=False, use_tc_tiling_on_sc=False
        ),
        scratch_types=[
            pltpu.VMEM((2, _NP, 8, 8, _LANES), jnp.float32),
            pltpu.VMEM((_HALF, _LANES), jnp.float32),
            pltpu.VMEM((_LANES, 2053), jnp.float32),
            pltpu.SemaphoreType.DMA((2,)),
        ],
    )
    def k(x_hbm, out_hbm, slab2, obuf, tbuf, sem):
        wid = lax.axis_index("s") * nc + lax.axis_index("c")
        g = wid // 2
        h = wid % 2
        ghi = g // 8
        glo = g % 8

        zeros = jnp.zeros((_LANES,), jnp.float32)

        def slab_copy(a, buf):
            return pltpu.make_async_copy(
                x_hbm.at[
                    pl.ds(a * _NP, _NP),
                    :,
                    ghi,
                    :,
                    pl.ds(glo * _LANES, _LANES),
                ],
                slab2.at[buf],
                sem.at[buf],
            )

        slab_copy(7, 0).start()
        slab_copy(8 * h, 1).start()

        def zbody(p_, _):
            obuf[p_, :] = zeros
            return 0

        lax.fori_loop(0, _HALF, zbody, 0, unroll=8)

        def accum(slab, b, base, i_list, ry_vecs):
            for blk in range(0, len(i_list), 2):
                pair = i_list[blk:blk + 2]
                vals = [
                    slab[b, i, j, :] * ry_vecs[i]
                    for i in pair
                    for j in range(_PSIZE)
                ]
                n = 0
                for i in pair:
                    for j in range(_PSIZE):
                        v = vals[n]
                        n += 1
                        if isinstance(b, int) and (
                            (b == 0 and j < 4) or (b == _NP - 1 and j >= 4)
                        ):
                            v = v + v
                        plsc.addupdate(
                            obuf.at[base + i * _IMAGE + 4 * b + j], v
                        )

        def emit_block(slab, a, hh, i_list, border_rows):
            base = (4 * a) * _IMAGE - hh * _HALF
            ry_vecs = {}
            for i in i_list:
                if border_rows:
                    y = 4 * a + i
                    ry = jnp.where(
                        (y < 4) | (y >= 60),
                        jnp.float32(0.5),
                        jnp.float32(0.25),
                    )
                else:
                    ry = jnp.float32(0.25)
                ry_vecs[i] = jnp.broadcast_to(ry, (_LANES,))

            accum(slab, 0, base, i_list, ry_vecs)

            def bbody(b, _):
                accum(slab, b, base, i_list, ry_vecs)
                return 0

            lax.fori_loop(1, _NP - 1, bbody, 0)
            accum(slab, _NP - 1, base, i_list, ry_vecs)

        slab_copy(7, 0).wait()

        @pl.when(h == 0)
        def _():
            emit_block(slab2.at[0], 7, 0, [0, 1, 2, 3], border_rows=False)

        @pl.when(h == 1)
        def _():
            emit_block(slab2.at[0], 7, 1, [4, 5, 6, 7], border_rows=False)

        slab_copy(8 * h + 1, 0).start()

        def main_body(s, _):
            a = 8 * h + s - 1
            buf = s % 2
            slab_copy(a, buf).wait()
            emit_block(slab2.at[buf], a, h, list(range(8)), border_rows=True)

            @pl.when(s <= 5)
            def _():
                slab_copy(a + 2, buf).start()

            return 0

        lax.fori_loop(1, 8, main_body, 0)

        lanes = lax.iota(jnp.int32, _LANES)

        def tbody(pix, _):
            v = obuf[pix, :]
            plsc.store_scatter(
                tbuf, [lanes, jnp.broadcast_to(pix, (_LANES,))], v
            )
            return 0

        lax.fori_loop(0, _HALF, tbody, 0, unroll=8)

        pltpu.sync_copy(
            tbuf.at[:, pl.ds(0, _HALF)],
            out_hbm.at[pl.ds(g * _LANES, _LANES), pl.ds(h * _HALF, _HALF)],
        )

    return k


def kernel(input_data):
    x5 = (
        input_data.transpose(1, 2, 0)
        .reshape(_NPATCH, 8, 8, 2, 128)
        .transpose(0, 1, 3, 2, 4)
    )
    out = _make_sc_kernel()(x5)
    return out.reshape(_BATCH, _CHANNELS, _IMAGE, _IMAGE)
